# Initial kernel scaffold; baseline (speedup 1.0000x reference)
import jax
import jax.numpy as jnp
from jax.experimental import pallas as pl


def kernel(x, pos_emb):
    raise NotImplementedError("write your pallas kernel here")


# speedup vs baseline: 1.6040x; 1.6040x over previous
import functools

import jax
import jax.numpy as jnp
from jax import lax
from jax.experimental import pallas as pl
from jax.experimental.pallas import tpu as pltpu
from jax.experimental.pallas import tpu_sc as plsc

B, S, D = 4, 8192, 2048
NC, NS = 2, 16
NW = NC * NS
ROWS_PER_W = S // NW
CH = 32
NCH = ROWS_PER_W // CH

_mesh = plsc.VectorSubcoreMesh(core_axis_name="c", subcore_axis_name="s")


@functools.partial(
    pl.kernel,
    mesh=_mesh,
    out_type=jax.ShapeDtypeStruct((B * S, D), jnp.float32),
    scratch_types=[
        pltpu.VMEM((CH, D), jnp.float32),
        pltpu.SemaphoreType.DMA,
        pltpu.SemaphoreType.DMA,
    ],
)
def _bcast_sc(pos_hbm, out_hbm, buf, gsem, ssem):
    wid = lax.axis_index("s") * NC + lax.axis_index("c")
    base0 = wid * ROWS_PER_W

    def body(c, carry):
        base = base0 + c * CH
        pltpu.async_copy(pos_hbm.at[pl.ds(base, CH)], buf, gsem).wait()
        copies = [
            pltpu.async_copy(buf, out_hbm.at[pl.ds(b * S + base, CH)], ssem)
            for b in range(B)
        ]
        for cp in copies:
            cp.wait()
        return carry

    lax.fori_loop(0, NCH, body, 0)


def kernel(x, pos_emb):
    del x
    out2d = _bcast_sc(pos_emb)
    return out2d.reshape(B, S, D)
